# Initial kernel scaffold; baseline (speedup 1.0000x reference)
import jax
import jax.numpy as jnp
from jax.experimental import pallas as pl


def kernel(x, embedding, feature_idx):
    raise NotImplementedError("write your pallas kernel here")


# speedup vs baseline: 16.9251x; 16.9251x over previous
import jax
import jax.numpy as jnp
from jax import lax
from jax.experimental import pallas as pl
from jax.experimental.pallas import tpu as pltpu
from jax.experimental.pallas import tpu_sc as plsc

B, F, C, D = 4096, 100, 1000, 64
NC, NS, L = 2, 16, 16
NW = NC * NS
ROWS = B * F
CHUNK = 128
NCHUNKS = ROWS // CHUNK
CPT = NCHUNKS // NW
K = 5
GROUPS = CPT // K
PERIOD = 25


def _body(emb_hbm, x_hbm, off_hbm, out_hbm, idx_v, rows_v, off_v, sem):
    wid = lax.axis_index("s") * NC + lax.axis_index("c")
    base = wid * CPT
    pltpu.sync_copy(off_hbm, off_v)

    def group(g, carry):
        c0 = base + g * K
        pltpu.sync_copy(x_hbm.at[pl.ds(c0 * CHUNK, K * CHUNK)], idx_v)
        gk = g * K
        for r in range(K):
            row = lax.rem(gk + r, PERIOD)
            for j in range(CHUNK // L):
                sl = pl.ds(r * CHUNK + j * L, L)
                idx_v[sl] = idx_v[sl] + off_v[row, pl.ds(j * L, L)]
        cps = [pltpu.async_copy(emb_hbm.at[idx_v.at[pl.ds(r * CHUNK, CHUNK)]],
                                rows_v.at[r], sem)
               for r in range(K)]
        for cp in cps:
            cp.wait()
        pltpu.sync_copy(rows_v, out_hbm.at[pl.ds(c0, K)])
        return carry

    lax.fori_loop(0, GROUPS, group, 0)


def kernel(x, embedding, feature_idx):
    x1d = x.reshape(ROWS)
    emb2d = embedding.reshape(F * C, D)
    pos = jnp.arange(PERIOD * CHUNK, dtype=jnp.int32)
    off = (feature_idx[pos % F] * C).astype(jnp.int32).reshape(PERIOD, CHUNK)
    mesh = plsc.VectorSubcoreMesh(core_axis_name="c", subcore_axis_name="s")
    k = pl.kernel(
        _body,
        mesh=mesh,
        compiler_params=pltpu.CompilerParams(use_tc_tiling_on_sc=False),
        out_type=jax.ShapeDtypeStruct((NCHUNKS, CHUNK, D), jnp.float32),
        scratch_types=[
            pltpu.VMEM((K * CHUNK,), jnp.int32),
            pltpu.VMEM((K, CHUNK, D), jnp.float32),
            pltpu.VMEM((PERIOD, CHUNK), jnp.int32),
            pltpu.SemaphoreType.DMA,
        ],
    )
    out = k(emb2d, x1d, off)
    return out.reshape(B, F, D)


# speedup vs baseline: 17.7244x; 1.0472x over previous
import jax
import jax.numpy as jnp
from jax import lax
from jax.experimental import pallas as pl
from jax.experimental.pallas import tpu as pltpu
from jax.experimental.pallas import tpu_sc as plsc

B, F, C, D = 4096, 100, 1000, 64
NC, NS, L = 2, 16, 16
NW = NC * NS
ROWS = B * F
CHUNK = 128
NCHUNKS = ROWS // CHUNK
CPT = NCHUNKS // NW
K = 5
GROUPS = CPT // K
SUPERS = GROUPS // 2
PERIOD = 25


def _body(emb_hbm, x_hbm, off_hbm, out_hbm, idx_v, rows_v, off_v,
          gsem0, gsem1, osem0, osem1):
    wid = lax.axis_index("s") * NC + lax.axis_index("c")
    base = wid * CPT
    pltpu.sync_copy(off_hbm, off_v)

    def load_compute(g, p):
        c0 = base + g * K
        pltpu.sync_copy(x_hbm.at[pl.ds(c0 * CHUNK, K * CHUNK)], idx_v.at[p])
        gk = g * K
        for r in range(K):
            row = lax.rem(gk + r, PERIOD)
            for j in range(CHUNK // L):
                sl = pl.ds(r * CHUNK + j * L, L)
                idx_v[p, sl] = idx_v[p, sl] + off_v[row, pl.ds(j * L, L)]

    def fire_gathers(p, gsem):
        for r in range(K):
            pltpu.async_copy(emb_hbm.at[idx_v.at[p, pl.ds(r * CHUNK, CHUNK)]],
                             rows_v.at[p, r], gsem)

    def drain_gathers(p, gsem):
        pltpu.make_async_copy(out_hbm.at[pl.ds(0, K)], rows_v.at[p], gsem).wait()

    def fire_store(g, p, osem):
        c0 = base + g * K
        pltpu.async_copy(rows_v.at[p], out_hbm.at[pl.ds(c0, K)], osem)

    def wait_store(p, osem):
        pltpu.make_async_copy(rows_v.at[p], out_hbm.at[pl.ds(0, K)], osem).wait()

    load_compute(0, 0)
    fire_gathers(0, gsem0)

    def super_body(s, carry):
        g0 = 2 * s
        g1 = g0 + 1
        g2 = g0 + 2

        @pl.when(s >= 1)
        def _():
            wait_store(1, osem1)
        load_compute(g1, 1)
        fire_gathers(1, gsem1)
        drain_gathers(0, gsem0)
        fire_store(g0, 0, osem0)

        @pl.when(s <= SUPERS - 2)
        def _():
            load_compute(g2, 0)
        wait_store(0, osem0)

        @pl.when(s <= SUPERS - 2)
        def _():
            fire_gathers(0, gsem0)
        drain_gathers(1, gsem1)
        fire_store(g1, 1, osem1)
        return carry

    lax.fori_loop(0, SUPERS, super_body, 0)
    wait_store(1, osem1)


def kernel(x, embedding, feature_idx):
    x1d = x.reshape(ROWS)
    emb2d = embedding.reshape(F * C, D)
    pos = jnp.arange(PERIOD * CHUNK, dtype=jnp.int32)
    off = (feature_idx[pos % F] * C).astype(jnp.int32).reshape(PERIOD, CHUNK)
    mesh = plsc.VectorSubcoreMesh(core_axis_name="c", subcore_axis_name="s")
    k = pl.kernel(
        _body,
        mesh=mesh,
        compiler_params=pltpu.CompilerParams(use_tc_tiling_on_sc=False),
        out_type=jax.ShapeDtypeStruct((NCHUNKS, CHUNK, D), jnp.float32),
        scratch_types=[
            pltpu.VMEM((2, K * CHUNK), jnp.int32),
            pltpu.VMEM((2, K, CHUNK, D), jnp.float32),
            pltpu.VMEM((PERIOD, CHUNK), jnp.int32),
            pltpu.SemaphoreType.DMA,
            pltpu.SemaphoreType.DMA,
            pltpu.SemaphoreType.DMA,
            pltpu.SemaphoreType.DMA,
        ],
    )
    out = k(emb2d, x1d, off)
    return out.reshape(B, F, D)
